# baseline (speedup 1.0000x reference)
import functools

import jax
import jax.numpy as jnp
from jax import lax
from jax.experimental import pallas as pl
from jax.experimental.pallas import tpu as pltpu
from jax.experimental.pallas import tpu_sc as plsc

_NUM_ENT = 1000000
_NUM_REL = 1000
_D = 64
_B = 16384

_NW = 32
_BPW = _B // _NW
_G = _BPW // 16

_NTC = 7813
_TPW = 245
_ENT_TAIL_J = 7812
_ENT_TAIL_L = _NUM_ENT - _ENT_TAIL_J * 128
_REL_TAIL_L = _NUM_REL - 7 * 128


def _transpose_block(chunk, stag, rowvecs):
    for c in range(_D):
        cvec = jnp.full((16,), c, jnp.int32)
        for m in range(8):
            v = chunk[c, pl.ds(m * 16, 16)]
            plsc.store_scatter(stag, [rowvecs[m], cvec], v)


def _convert_body(ent_t, ent_tail, ent_lin,
                  ch0, ch1, st0, st1, seml0, seml1, sems0, sems1):
    wid = lax.axis_index("s") * 2 + lax.axis_index("c")
    j_base = wid * _TPW
    iota16 = lax.iota(jnp.int32, 16)
    rowvecs = [m * 16 + iota16 for m in range(8)]

    chs = (ch0, ch1)
    sts = (st0, st1)
    semls = (seml0, seml1)
    semss = (sems0, sems1)

    def issue_load(p, j):
        src = ent_t.at[:, pl.ds(pl.multiple_of(j * 128, 128), 128)]
        pltpu.async_copy(src, chs[p], semls[p])

    def wait_load(p):
        pltpu.make_async_copy(ent_t.at[:, pl.ds(0, 128)], chs[p],
                              semls[p]).wait()

    def issue_store(p, j):
        dst = ent_lin.at[pl.ds(pl.multiple_of(j * 128, 128), 128), :]
        pltpu.async_copy(sts[p], dst, semss[p])

    def wait_store(p):
        pltpu.make_async_copy(ent_lin.at[pl.ds(0, 128), :], sts[p],
                              semss[p]).wait()

    npairs = (_TPW + 1) // 2

    @pl.when(j_base < _ENT_TAIL_J)
    def _():
        issue_load(0, j_base)

    def pair_body(i, carry):
        g0 = j_base + 2 * i

        def half(p, g):
            @pl.when(g < _ENT_TAIL_J)
            def _():
                wait_load(p)

                @pl.when(i > 0)
                def _():
                    wait_store(p)

                _transpose_block(chs[p], sts[p], rowvecs)
                issue_store(p, g)

        @pl.when(g0 + 1 < _ENT_TAIL_J)
        def _():
            issue_load(1, g0 + 1)

        half(0, g0)

        @pl.when((i < npairs - 1) & (g0 + 2 < _ENT_TAIL_J))
        def _():
            issue_load(0, g0 + 2)

        half(1, g0 + 1)
        return carry

    lax.fori_loop(0, npairs, pair_body, jnp.int32(0))
    wait_store(0)
    wait_store(1)

    @pl.when(wid == 31)
    def _():
        pltpu.sync_copy(ent_t.at[:, pl.ds((_ENT_TAIL_J - 1) * 128, 128)], ch0)
        pltpu.sync_copy(ent_tail, ch1)
        for c in range(_D):
            cvec = jnp.full((16,), c, jnp.int32)
            for m in range(4):
                v = ch0[c, pl.ds(64 + m * 16, 16)]
                plsc.store_scatter(st0, [rowvecs[m], cvec], v)
            for m in range(4):
                v = ch1[c, pl.ds(m * 16, 16)]
                plsc.store_scatter(st0, [rowvecs[4 + m], cvec], v)
        pltpu.sync_copy(
            st0, ent_lin.at[pl.ds((_ENT_TAIL_J - 1) * 128 + 64, 128), :])


def _gather_body(hs_hbm, rs_hbm, ts_hbm, ent_hbm, rel_hbm, out_hbm,
                 hs_v, rs_v, ts_v, hb, rb, tb, o_v, sem0, sem1):
    wid = lax.axis_index("s") * 2 + lax.axis_index("c")
    base = wid * _BPW

    pltpu.sync_copy(hs_hbm.at[pl.ds(base, _BPW)], hs_v)
    pltpu.sync_copy(rs_hbm.at[pl.ds(base, _BPW)], rs_v)
    pltpu.sync_copy(ts_hbm.at[pl.ds(base, _BPW)], ts_v)

    sems = (sem0, sem1)

    def issue(p, g):
        sem = sems[p]
        hv = hs_v[pl.ds(g * 16, 16)]
        rv = rs_v[pl.ds(g * 16, 16)]
        tv = ts_v[pl.ds(g * 16, 16)]
        for k in range(16):
            pltpu.async_copy(ent_hbm.at[hv[k]], hb.at[p, k, pl.ds(0, _D)], sem)
            pltpu.async_copy(rel_hbm.at[rv[k]], rb.at[p, k, pl.ds(0, _D)], sem)
            pltpu.async_copy(ent_hbm.at[tv[k]], tb.at[p, k, pl.ds(0, _D)], sem)

    def drain(p):
        sem = sems[p]
        for k in range(16):
            for buf, src in ((hb, ent_hbm), (rb, rel_hbm), (tb, ent_hbm)):
                pltpu.make_async_copy(
                    src.at[0], buf.at[p, k, pl.ds(0, _D)], sem
                ).wait()

    iota16 = lax.iota(jnp.int32, 16)
    half = jnp.float32(0.5)
    threehalf = jnp.float32(1.5)
    magic = jnp.int32(0x5F3759DF)

    def compute(p, g):
        pvec = jnp.full((16,), p, jnp.int32)

        def j_body(j, acc):
            col = jnp.full((16,), j, jnp.int32)
            hvv = plsc.load_gather(hb, [pvec, iota16, col])
            rvv = plsc.load_gather(rb, [pvec, iota16, col])
            tvv = plsc.load_gather(tb, [pvec, iota16, col])
            d = (hvv + rvv) - tvv
            return acc + d * d

        acc = lax.fori_loop(0, _D, j_body, jnp.zeros((16,), jnp.float32))

        bits = lax.bitcast_convert_type(acc, jnp.int32)
        y = lax.bitcast_convert_type(magic - (bits >> 1), jnp.float32)
        hx = half * acc
        for _ in range(3):
            y = y * (threehalf - hx * y * y)
        o_v[pl.ds(g * 16, 16)] = acc * y

    issue(0, 0)

    def pair_body(i, carry):
        g0 = 2 * i
        issue(1, g0 + 1)
        drain(0)
        compute(0, g0)

        @pl.when(i < _G // 2 - 1)
        def _():
            issue(0, g0 + 2)

        drain(1)
        compute(1, g0 + 1)
        return carry

    lax.fori_loop(0, _G // 2, pair_body, jnp.int32(0))
    pltpu.sync_copy(o_v, out_hbm.at[pl.ds(base, _BPW)])


@jax.jit
def _transe_call(hs, rs, ts, ent_t, ent_tail, rel_embs):
    mesh = plsc.VectorSubcoreMesh(core_axis_name="c", subcore_axis_name="s")
    convert = functools.partial(
        pl.kernel,
        mesh=mesh,
        out_type=jax.ShapeDtypeStruct((_NUM_ENT, _D), jnp.float32),
        compiler_params=pltpu.CompilerParams(needs_layout_passes=False),
        scratch_types=[
            pltpu.VMEM((_D, 128), jnp.float32),
            pltpu.VMEM((_D, 128), jnp.float32),
            pltpu.VMEM((128, _D), jnp.float32),
            pltpu.VMEM((128, _D), jnp.float32),
            pltpu.SemaphoreType.DMA,
            pltpu.SemaphoreType.DMA,
            pltpu.SemaphoreType.DMA,
            pltpu.SemaphoreType.DMA,
        ],
    )(_convert_body)
    ent_lin = convert(ent_t, ent_tail)

    gather = functools.partial(
        pl.kernel,
        mesh=mesh,
        out_type=jax.ShapeDtypeStruct((_B,), jnp.float32),
        compiler_params=pltpu.CompilerParams(needs_layout_passes=False),
        scratch_types=[
            pltpu.VMEM((_BPW,), jnp.int32),
            pltpu.VMEM((_BPW,), jnp.int32),
            pltpu.VMEM((_BPW,), jnp.int32),
            pltpu.VMEM((2, 16, 128), jnp.float32),
            pltpu.VMEM((2, 16, 128), jnp.float32),
            pltpu.VMEM((2, 16, 128), jnp.float32),
            pltpu.VMEM((_BPW,), jnp.float32),
            pltpu.SemaphoreType.DMA,
            pltpu.SemaphoreType.DMA,
        ],
    )(_gather_body)
    return gather(hs, rs, ts, ent_lin, rel_embs)


def kernel(hs, rs, ts, ent_embs, rel_embs):
    tail = jnp.pad(ent_embs[_ENT_TAIL_J * 128:].T, ((0, 0), (0, 64)))
    out = _transe_call(hs, rs, ts, ent_embs.T, tail, rel_embs)
    return out.reshape(-1, 1)


# speedup vs baseline: 3.1988x; 3.1988x over previous
import functools

import jax
import jax.numpy as jnp
from jax import lax
from jax.experimental import pallas as pl
from jax.experimental.pallas import tpu as pltpu
from jax.experimental.pallas import tpu_sc as plsc

_NUM_ENT = 1000000
_NUM_REL = 1000
_D = 64
_B = 16384

_NW = 32
_BPW = _B // _NW
_G = _BPW // 16


def _transe_body(hs_hbm, rs_hbm, ts_hbm, ent_hbm, rel_hbm, out_hbm,
                 hs_v, rs_v, ts_v, hb, rb, tb, o_v, sem0, sem1):
    wid = lax.axis_index("s") * 2 + lax.axis_index("c")
    base = wid * _BPW

    pltpu.sync_copy(hs_hbm.at[pl.ds(base, _BPW)], hs_v)
    pltpu.sync_copy(rs_hbm.at[pl.ds(base, _BPW)], rs_v)
    pltpu.sync_copy(ts_hbm.at[pl.ds(base, _BPW)], ts_v)

    sems = (sem0, sem1)

    def issue(p, g):
        sem = sems[p]
        hv = hs_v[pl.ds(g * 16, 16)]
        rv = rs_v[pl.ds(g * 16, 16)]
        tv = ts_v[pl.ds(g * 16, 16)]
        for k in range(16):
            pltpu.async_copy(ent_hbm.at[hv[k]], hb.at[p, k, pl.ds(0, _D)], sem)
            pltpu.async_copy(rel_hbm.at[rv[k]], rb.at[p, k, pl.ds(0, _D)], sem)
            pltpu.async_copy(ent_hbm.at[tv[k]], tb.at[p, k, pl.ds(0, _D)], sem)

    def drain(p):
        sem = sems[p]
        for k in range(16):
            for buf, src in ((hb, ent_hbm), (rb, rel_hbm), (tb, ent_hbm)):
                pltpu.make_async_copy(
                    src.at[0], buf.at[p, k, pl.ds(0, _D)], sem
                ).wait()

    iota16 = lax.iota(jnp.int32, 16)
    half = jnp.float32(0.5)
    threehalf = jnp.float32(1.5)
    magic = jnp.int32(0x5F3759DF)

    def compute(p, g):
        pvec = jnp.full((16,), p, jnp.int32)

        def j_body(j, accs):
            a0, a1, a2, a3 = accs
            ds_ = []
            for q in range(4):
                col = jnp.full((16,), 4 * j + q, jnp.int32)
                ds_.append(plsc.load_gather(hb, [pvec, iota16, col])
                           + plsc.load_gather(rb, [pvec, iota16, col])
                           - plsc.load_gather(tb, [pvec, iota16, col]))
            return (a0 + ds_[0] * ds_[0], a1 + ds_[1] * ds_[1],
                    a2 + ds_[2] * ds_[2], a3 + ds_[3] * ds_[3])

        z = jnp.zeros((16,), jnp.float32)
        a0, a1, a2, a3 = lax.fori_loop(0, _D // 4, j_body, (z, z, z, z))
        acc = (a0 + a1) + (a2 + a3)

        bits = lax.bitcast_convert_type(acc, jnp.int32)
        y = lax.bitcast_convert_type(magic - (bits >> 1), jnp.float32)
        hx = half * acc
        for _ in range(3):
            y = y * (threehalf - hx * y * y)
        o_v[pl.ds(g * 16, 16)] = acc * y

    issue(0, 0)

    def pair_body(i, carry):
        g0 = 2 * i
        issue(1, g0 + 1)
        drain(0)
        compute(0, g0)

        @pl.when(i < _G // 2 - 1)
        def _():
            issue(0, g0 + 2)

        drain(1)
        compute(1, g0 + 1)
        return carry

    lax.fori_loop(0, _G // 2, pair_body, jnp.int32(0))
    pltpu.sync_copy(o_v, out_hbm.at[pl.ds(base, _BPW)])


@jax.jit
def _transe_call(hs, rs, ts, ent_embs, rel_embs):
    mesh = plsc.VectorSubcoreMesh(core_axis_name="c", subcore_axis_name="s")
    fn = functools.partial(
        pl.kernel,
        mesh=mesh,
        out_type=jax.ShapeDtypeStruct((_B,), jnp.float32),
        compiler_params=pltpu.CompilerParams(needs_layout_passes=False),
        scratch_types=[
            pltpu.VMEM((_BPW,), jnp.int32),
            pltpu.VMEM((_BPW,), jnp.int32),
            pltpu.VMEM((_BPW,), jnp.int32),
            pltpu.VMEM((2, 16, 128), jnp.float32),
            pltpu.VMEM((2, 16, 128), jnp.float32),
            pltpu.VMEM((2, 16, 128), jnp.float32),
            pltpu.VMEM((_BPW,), jnp.float32),
            pltpu.SemaphoreType.DMA,
            pltpu.SemaphoreType.DMA,
        ],
    )(_transe_body)
    return fn(hs, rs, ts, ent_embs, rel_embs)


def kernel(hs, rs, ts, ent_embs, rel_embs):
    out = _transe_call(hs, rs, ts, ent_embs, rel_embs)
    return out.reshape(-1, 1)
